# baseline (speedup 1.0000x reference)
import functools
import math

import jax
import jax.numpy as jnp
from jax import lax
from jax.experimental import pallas as pl
from jax.experimental.pallas import tpu as pltpu
from jax.experimental.pallas import tpu_sc as plsc

_NUM_CORES = 2
_NUM_SUBCORES = 16
_NUM_WORKERS = _NUM_CORES * _NUM_SUBCORES
_LANES = 16
_BBLK = 128
_K = 4


def _make_emb_kernel(batch, seq, d):
    scale = math.sqrt(d)
    mesh = plsc.VectorSubcoreMesh(core_axis_name="c", subcore_axis_name="s")

    gbufs = [pltpu.VMEM((_BBLK, d), jnp.float32) for _ in range(_K)]
    tbufs = [pltpu.VMEM((d // 8, 8, _BBLK), jnp.float32) for _ in range(_K)]

    @functools.partial(
        pl.kernel,
        out_type=jax.ShapeDtypeStruct(
            (seq, d // 8, batch // _BBLK, 8, _BBLK), jnp.float32
        ),
        mesh=mesh,
        compiler_params=pltpu.CompilerParams(use_tc_tiling_on_sc=False, needs_layout_passes=False),
        scratch_types=[pltpu.VMEM((seq, _BBLK), jnp.int32)]
        + gbufs
        + tbufs
        + [pltpu.SemaphoreType.DMA, pltpu.SemaphoreType.DMA],
    )
    def emb(table_hbm, idx_hbm, q_hbm, idx_v, *bufs_and_sems):
        gb = bufs_and_sems[:_K]
        tb = bufs_and_sems[_K : 2 * _K]
        gsem, wsem = bufs_and_sems[2 * _K], bufs_and_sems[2 * _K + 1]
        w = lax.axis_index("s") * _NUM_CORES + lax.axis_index("c")
        pltpu.sync_copy(idx_hbm.at[:, w], idx_v)
        lanes = lax.iota(jnp.int32, _LANES)

        def body(i, carry):
            s0 = i * _K
            handles = [
                pltpu.async_copy(
                    table_hbm.at[idx_v.at[s0 + k]], gb[k], gsem
                )
                for k in range(_K)
            ]
            for k in range(_K):
                handles[k].wait()

                def tloop(c, carry2, g=gb[k], t=tb[k]):
                    cvec = jnp.full((_LANES,), 0, jnp.int32) + c
                    for bg in range(_BBLK // _LANES):
                        vals = plsc.load_gather(
                            g, [bg * _LANES + lanes, cvec]
                        )
                        t[c >> 3, c & 7, pl.ds(bg * _LANES, _LANES)] = (
                            vals * scale
                        )
                    return carry2

                lax.fori_loop(0, d, tloop, 0)

                @pl.when(i > 0)
                def _(t=tb[k]):
                    pltpu.make_async_copy(q_hbm.at[0, :, w], t, wsem).wait()

                pltpu.async_copy(tb[k], q_hbm.at[s0 + k, :, w], wsem)
            return carry

        lax.fori_loop(0, seq // _K, body, 0)
        for _ in range(_K):
            pltpu.make_async_copy(q_hbm.at[0, :, w], tb[0], wsem).wait()

    return emb


def kernel(x, table):
    batch, seq = x.shape
    vocab, d = table.shape
    x3 = x.T.astype(jnp.int32).reshape(seq, batch // _BBLK, _BBLK)
    emb = _make_emb_kernel(batch, seq, d)
    q = emb(table, x3)
    out = q.transpose(2, 4, 0, 1, 3).reshape(batch, seq, d)
    return out
